# baseline (speedup 1.0000x reference)
import functools

import jax
import jax.numpy as jnp
from jax import lax
from jax.experimental import pallas as pl
from jax.experimental.pallas import tpu as pltpu
from jax.experimental.pallas import tpu_sc as plsc

VOCAB = 1000000
EMB_DIM = 64
B = 4096
L = 200
EPS = 1e-12

NW = 32
NB = B // NW
TROWS = VOCAB // 2

GBASES = tuple(range(0, 192, 16)) + (184,)
QUADS = ((0, 16, 32, 48), (64, 80, 96, 112), (128, 144, 160, 176), (184,))


def _rsqrt(x):
    xi = lax.bitcast_convert_type(x, jnp.int32)
    y = lax.bitcast_convert_type(jnp.int32(0x5F3759DF) - (xi >> 1),
                                 jnp.float32)
    for _ in range(3):
        y = y * (1.5 - 0.5 * x * y * y)
    return y


def _ln_groups(in_ref, out_ref, gamma_ref, beta_ref, idx_ref, ibase, bases):
    iota16 = lax.iota(jnp.int32, 16)
    row_ids = [g + iota16 for g in bases]
    pvs = [(idx_ref[pl.ds(ibase + g, 16)] & 1) << 6 for g in bases]
    ng = len(bases)

    def ph1(jb, carry):
        accs = list(carry)
        for jj in range(16):
            colj = jb * 16 + jj
            for g in range(ng):
                v = plsc.load_gather(in_ref, [row_ids[g], pvs[g] + colj])
                accs[2 * g] = accs[2 * g] + v
                accs[2 * g + 1] = accs[2 * g + 1] + v * v
        return tuple(accs)

    zero = jnp.zeros((16,), jnp.float32)
    accs = lax.fori_loop(0, 4, ph1, (zero,) * (2 * ng))

    c1s, c2s = [], []
    for g in range(ng):
        mean = accs[2 * g] * (1.0 / EMB_DIM)
        var = accs[2 * g + 1] * (1.0 / EMB_DIM) - mean * mean
        rstd = _rsqrt(var + EPS)
        c1s.append(rstd)
        c2s.append(-mean * rstd)

    def ph3(jb, carry):
        for jj in range(16):
            colj = jb * 16 + jj
            cvec = jnp.broadcast_to(colj, (16,)).astype(jnp.int32)
            gj = plsc.load_gather(gamma_ref, [cvec])
            bj = plsc.load_gather(beta_ref, [cvec])
            for g in range(ng):
                v = plsc.load_gather(in_ref, [row_ids[g], pvs[g] + colj])
                o = (v * c1s[g] + c2s[g]) * gj + bj
                plsc.store_scatter(out_ref, [row_ids[g], cvec], o)
        return carry

    lax.fori_loop(0, 4, ph3, 0)


def _make_kernel():
    mesh = plsc.VectorSubcoreMesh(core_axis_name="c", subcore_axis_name="s")

    @functools.partial(
        pl.kernel,
        mesh=mesh,
        out_type=jax.ShapeDtypeStruct((B * L, EMB_DIM), jnp.float32),
        compiler_params=pltpu.CompilerParams(
            use_tc_tiling_on_sc=False,
            needs_layout_passes=False,
        ),
        scratch_types=[
            pltpu.VMEM((NB * L,), jnp.int32),
            pltpu.VMEM((L, 128), jnp.float32),
            pltpu.VMEM((L, 128), jnp.float32),
            pltpu.VMEM((L, EMB_DIM), jnp.float32),
            pltpu.VMEM((L, EMB_DIM), jnp.float32),
            pltpu.VMEM((EMB_DIM,), jnp.float32),
            pltpu.VMEM((EMB_DIM,), jnp.float32),
            pltpu.SemaphoreType.DMA,
            pltpu.SemaphoreType.DMA,
            pltpu.SemaphoreType.DMA,
            pltpu.SemaphoreType.DMA,
        ],
    )
    def kern(ids_hbm, table_hbm, gamma_hbm, beta_hbm, out_hbm,
             idx_v, in0, in1, out0, out1, gamma_v, beta_v,
             gsem0, gsem1, osem0, osem1):
        wid = lax.axis_index("s") * 2 + lax.axis_index("c")
        wrow = wid * NB * L

        pltpu.sync_copy(gamma_hbm, gamma_v)
        pltpu.sync_copy(beta_hbm, beta_v)
        pltpu.sync_copy(ids_hbm.at[wid], idx_v)

        ins = (in0, in1)
        outs = (out0, out1)
        gsems = (gsem0, gsem1)
        osems = (osem0, osem1)

        def gather_start(c, b):
            for g in GBASES:
                gidx = idx_v[pl.ds(c * L + g, 16)] >> 1
                pltpu.async_copy(table_hbm.at[gidx],
                                 ins[b].at[pl.ds(g, 16)], gsems[b])

        def gather_wait(c, b):
            for g in GBASES:
                gidx = idx_v[pl.ds(c * L + g, 16)] >> 1
                pltpu.make_async_copy(table_hbm.at[gidx],
                                      ins[b].at[pl.ds(g, 16)],
                                      gsems[b]).wait()

        def out_start(c, b):
            pltpu.async_copy(outs[b],
                             out_hbm.at[pl.ds(wrow + c * L, L)], osems[b])

        def out_wait(c, b):
            pltpu.make_async_copy(outs[b],
                                  out_hbm.at[pl.ds(wrow + c * L, L)],
                                  osems[b]).wait()

        gather_start(0, 0)
        gather_start(1, 1)

        def body(i, carry):
            for b in range(2):
                c = 2 * i + b
                gather_wait(c, b)

                @pl.when(c >= 2)
                def _():
                    out_wait(c - 2, b)

                for quad in QUADS:
                    _ln_groups(ins[b], outs[b], gamma_v, beta_v,
                               idx_v, c * L, quad)

                out_start(c, b)

                @pl.when(c + 2 < NB)
                def _():
                    gather_start(c + 2, b)
            return carry

        lax.fori_loop(0, NB // 2, body, 0)

        out_wait(NB - 2, 0)
        out_wait(NB - 1, 1)

    return kern


_KERNEL = _make_kernel()


@jax.jit
def kernel(input_ids, table, ln_gamma, ln_beta):
    ids = input_ids.reshape(NW, NB * L)
    tv = table.reshape(TROWS, 128)
    out = _KERNEL(ids, tv, ln_gamma, ln_beta)
    return out.reshape(B, L, EMB_DIM)


# speedup vs baseline: 1.7701x; 1.7701x over previous
import functools

import jax
import jax.numpy as jnp
from jax import lax
from jax.experimental import pallas as pl
from jax.experimental.pallas import tpu as pltpu
from jax.experimental.pallas import tpu_sc as plsc

VOCAB = 1000000
EMB_DIM = 64
B = 4096
L = 200
EPS = 1e-12

NW = 32
NB = B // NW
TROWS = VOCAB // 2

GBASES = tuple(range(0, 192, 16)) + (184,)
QUADS = ((0, 16, 32, 48), (64, 80, 96, 112), (128, 144, 160, 176), (184,))


def _rsqrt(x):
    xi = lax.bitcast_convert_type(x, jnp.int32)
    y = lax.bitcast_convert_type(jnp.int32(0x5F3759DF) - (xi >> 1),
                                 jnp.float32)
    for _ in range(3):
        y = y * (1.5 - 0.5 * x * y * y)
    return y


def _ln_16rows(in_ref, out_ref, gamma_ref, beta_ref, idx_ref, ibase, gbase):
    gs = [gamma_ref[pl.ds(16 * k, 16)] for k in range(4)]
    bs = [beta_ref[pl.ds(16 * k, 16)] for k in range(4)]
    pvv = (idx_ref[pl.ds(ibase + gbase, 16)] & 1) << 6
    for k in range(16):
        r = gbase + k
        ofs = pvv[k]
        vs = [in_ref[r, pl.ds(ofs + 16 * q, 16)] for q in range(4)]
        s = (vs[0] + vs[1]) + (vs[2] + vs[3])
        sq = (vs[0] * vs[0] + vs[1] * vs[1]) + (vs[2] * vs[2]
                                                + vs[3] * vs[3])
        tot = plsc.cumsum(s)[15]
        tot2 = plsc.cumsum(sq)[15]
        mean = tot * (1.0 / EMB_DIM)
        var = tot2 * (1.0 / EMB_DIM) - mean * mean
        rstd = _rsqrt(var + EPS)
        c1 = rstd
        c2 = -mean * rstd
        for q in range(4):
            out_ref[r, pl.ds(16 * q, 16)] = (vs[q] * c1 + c2) * gs[q] + bs[q]


def _make_kernel():
    mesh = plsc.VectorSubcoreMesh(core_axis_name="c", subcore_axis_name="s")

    @functools.partial(
        pl.kernel,
        mesh=mesh,
        out_type=jax.ShapeDtypeStruct((B * L, EMB_DIM), jnp.float32),
        compiler_params=pltpu.CompilerParams(
            use_tc_tiling_on_sc=False,
            needs_layout_passes=False,
        ),
        scratch_types=[
            pltpu.VMEM((NB * L,), jnp.int32),
            pltpu.VMEM((L, 128), jnp.float32),
            pltpu.VMEM((L, 128), jnp.float32),
            pltpu.VMEM((L, EMB_DIM), jnp.float32),
            pltpu.VMEM((L, EMB_DIM), jnp.float32),
            pltpu.VMEM((EMB_DIM,), jnp.float32),
            pltpu.VMEM((EMB_DIM,), jnp.float32),
            pltpu.SemaphoreType.DMA,
            pltpu.SemaphoreType.DMA,
            pltpu.SemaphoreType.DMA,
            pltpu.SemaphoreType.DMA,
        ],
    )
    def kern(ids_hbm, table_hbm, gamma_hbm, beta_hbm, out_hbm,
             idx_v, in0, in1, out0, out1, gamma_v, beta_v,
             gsem0, gsem1, osem0, osem1):
        wid = lax.axis_index("s") * 2 + lax.axis_index("c")
        wrow = wid * NB * L

        pltpu.sync_copy(gamma_hbm, gamma_v)
        pltpu.sync_copy(beta_hbm, beta_v)
        pltpu.sync_copy(ids_hbm.at[wid], idx_v)

        ins = (in0, in1)
        outs = (out0, out1)
        gsems = (gsem0, gsem1)
        osems = (osem0, osem1)

        def gather_start(c, b):
            for g in GBASES:
                gidx = idx_v[pl.ds(c * L + g, 16)] >> 1
                pltpu.async_copy(table_hbm.at[gidx],
                                 ins[b].at[pl.ds(g, 16)], gsems[b])

        def gather_wait(c, b):
            for g in GBASES:
                gidx = idx_v[pl.ds(c * L + g, 16)] >> 1
                pltpu.make_async_copy(table_hbm.at[gidx],
                                      ins[b].at[pl.ds(g, 16)],
                                      gsems[b]).wait()

        def out_start(c, b):
            pltpu.async_copy(outs[b],
                             out_hbm.at[pl.ds(wrow + c * L, L)], osems[b])

        def out_wait(c, b):
            pltpu.make_async_copy(outs[b],
                                  out_hbm.at[pl.ds(wrow + c * L, L)],
                                  osems[b]).wait()

        gather_start(0, 0)
        gather_start(1, 1)

        def body(i, carry):
            for b in range(2):
                c = 2 * i + b
                gather_wait(c, b)

                @pl.when(c >= 2)
                def _():
                    out_wait(c - 2, b)

                def grp(gi, carry2):
                    _ln_16rows(ins[b], outs[b], gamma_v, beta_v,
                               idx_v, c * L, gi * 16)
                    return carry2

                lax.fori_loop(0, 12, grp, 0)
                _ln_16rows(ins[b], outs[b], gamma_v, beta_v,
                           idx_v, c * L, 184)

                out_start(c, b)

                @pl.when(c + 2 < NB)
                def _():
                    gather_start(c + 2, b)
            return carry

        lax.fori_loop(0, NB // 2, body, 0)

        out_wait(NB - 2, 0)
        out_wait(NB - 1, 1)

    return kern


_KERNEL = _make_kernel()


@jax.jit
def kernel(input_ids, table, ln_gamma, ln_beta):
    ids = input_ids.reshape(NW, NB * L)
    tv = table.reshape(TROWS, 128)
    out = _KERNEL(ids, tv, ln_gamma, ln_beta)
    return out.reshape(B, L, EMB_DIM)


# speedup vs baseline: 2.6343x; 1.4882x over previous
import functools

import jax
import jax.numpy as jnp
from jax import lax
from jax.experimental import pallas as pl
from jax.experimental.pallas import tpu as pltpu
from jax.experimental.pallas import tpu_sc as plsc

VOCAB = 1000000
EMB_DIM = 64
B = 4096
L = 200
EPS = 1e-12

NW = 32
NB = B // NW
TROWS = VOCAB // 2

GBASES = tuple(range(0, 192, 16)) + (184,)
QUADS = ((0, 16, 32, 48), (64, 80, 96, 112), (128, 144, 160, 176), (184,))


def _rsqrt(x):
    xi = lax.bitcast_convert_type(x, jnp.int32)
    y = lax.bitcast_convert_type(jnp.int32(0x5F3759DF) - (xi >> 1),
                                 jnp.float32)
    for _ in range(3):
        y = y * (1.5 - 0.5 * x * y * y)
    return y


def _ln_16rows(in_ref, out_ref, gamma_ref, beta_ref, idx_ref, ibase, gbase):
    gs = [gamma_ref[pl.ds(16 * k, 16)] for k in range(4)]
    bs = [beta_ref[pl.ds(16 * k, 16)] for k in range(4)]
    pvv = (idx_ref[pl.ds(ibase + gbase, 16)] & 1) << 6
    for k4 in range(4):
        rows = [gbase + 4 * k4 + k for k in range(4)]
        ofss = [pvv[4 * k4 + k] for k in range(4)]
        vss = [[in_ref[rows[k], pl.ds(ofss[k] + 16 * q, 16)]
                for q in range(4)] for k in range(4)]
        ss = [(vs[0] + vs[1]) + (vs[2] + vs[3]) for vs in vss]
        sqs = [(vs[0] * vs[0] + vs[1] * vs[1])
               + (vs[2] * vs[2] + vs[3] * vs[3]) for vs in vss]
        tots = [plsc.cumsum(s)[15] for s in ss]
        tot2s = [plsc.cumsum(sq)[15] for sq in sqs]
        c1s, c2s = [], []
        for k in range(4):
            mean = tots[k] * (1.0 / EMB_DIM)
            var = tot2s[k] * (1.0 / EMB_DIM) - mean * mean
            rstd = _rsqrt(var + EPS)
            c1s.append(rstd)
            c2s.append(-mean * rstd)
        for k in range(4):
            for q in range(4):
                out_ref[rows[k], pl.ds(16 * q, 16)] = (
                    (vss[k][q] * c1s[k] + c2s[k]) * gs[q] + bs[q])


def _make_kernel():
    mesh = plsc.VectorSubcoreMesh(core_axis_name="c", subcore_axis_name="s")

    @functools.partial(
        pl.kernel,
        mesh=mesh,
        out_type=jax.ShapeDtypeStruct((B * L, EMB_DIM), jnp.float32),
        compiler_params=pltpu.CompilerParams(
            use_tc_tiling_on_sc=False,
            needs_layout_passes=False,
        ),
        scratch_types=[
            pltpu.VMEM((NB * L,), jnp.int32),
            pltpu.VMEM((L, 128), jnp.float32),
            pltpu.VMEM((L, 128), jnp.float32),
            pltpu.VMEM((L, EMB_DIM), jnp.float32),
            pltpu.VMEM((L, EMB_DIM), jnp.float32),
            pltpu.VMEM((EMB_DIM,), jnp.float32),
            pltpu.VMEM((EMB_DIM,), jnp.float32),
            pltpu.SemaphoreType.DMA,
            pltpu.SemaphoreType.DMA,
            pltpu.SemaphoreType.DMA,
            pltpu.SemaphoreType.DMA,
        ],
    )
    def kern(ids_hbm, table_hbm, gamma_hbm, beta_hbm, out_hbm,
             idx_v, in0, in1, out0, out1, gamma_v, beta_v,
             gsem0, gsem1, osem0, osem1):
        wid = lax.axis_index("s") * 2 + lax.axis_index("c")
        wrow = wid * NB * L

        pltpu.sync_copy(gamma_hbm, gamma_v)
        pltpu.sync_copy(beta_hbm, beta_v)
        pltpu.sync_copy(ids_hbm.at[wid], idx_v)

        ins = (in0, in1)
        outs = (out0, out1)
        gsems = (gsem0, gsem1)
        osems = (osem0, osem1)

        def gather_start(c, b):
            for g in GBASES:
                gidx = idx_v[pl.ds(c * L + g, 16)] >> 1
                pltpu.async_copy(table_hbm.at[gidx],
                                 ins[b].at[pl.ds(g, 16)], gsems[b])

        def gather_wait(c, b):
            for g in GBASES:
                gidx = idx_v[pl.ds(c * L + g, 16)] >> 1
                pltpu.make_async_copy(table_hbm.at[gidx],
                                      ins[b].at[pl.ds(g, 16)],
                                      gsems[b]).wait()

        def out_start(c, b):
            pltpu.async_copy(outs[b],
                             out_hbm.at[pl.ds(wrow + c * L, L)], osems[b])

        def out_wait(c, b):
            pltpu.make_async_copy(outs[b],
                                  out_hbm.at[pl.ds(wrow + c * L, L)],
                                  osems[b]).wait()

        gather_start(0, 0)
        gather_start(1, 1)

        def body(i, carry):
            for b in range(2):
                c = 2 * i + b
                gather_wait(c, b)

                @pl.when(c >= 2)
                def _():
                    out_wait(c - 2, b)

                def grp(gi, carry2):
                    _ln_16rows(ins[b], outs[b], gamma_v, beta_v,
                               idx_v, c * L, gi * 16)
                    return carry2

                lax.fori_loop(0, 12, grp, 0)
                _ln_16rows(ins[b], outs[b], gamma_v, beta_v,
                           idx_v, c * L, 184)

                out_start(c, b)

                @pl.when(c + 2 < NB)
                def _():
                    gather_start(c + 2, b)
            return carry

        lax.fori_loop(0, NB // 2, body, 0)

        out_wait(NB - 2, 0)
        out_wait(NB - 1, 1)

    return kern


_KERNEL = _make_kernel()


@jax.jit
def kernel(input_ids, table, ln_gamma, ln_beta):
    ids = input_ids.reshape(NW, NB * L)
    tv = table.reshape(TROWS, 128)
    out = _KERNEL(ids, tv, ln_gamma, ln_beta)
    return out.reshape(B, L, EMB_DIM)


# speedup vs baseline: 2.9352x; 1.1142x over previous
import functools

import jax
import jax.numpy as jnp
from jax import lax
from jax.experimental import pallas as pl
from jax.experimental.pallas import tpu as pltpu
from jax.experimental.pallas import tpu_sc as plsc

VOCAB = 1000000
EMB_DIM = 64
B = 4096
L = 200
EPS = 1e-12

NW = 32
NB = B // NW
TROWS = VOCAB // 2

GBASES = tuple(range(0, 192, 16)) + (184,)
QUADS = ((0, 16, 32, 48), (64, 80, 96, 112), (128, 144, 160, 176), (184,))


def _rsqrt(x):
    xi = lax.bitcast_convert_type(x, jnp.int32)
    y = lax.bitcast_convert_type(jnp.int32(0x5F3759DF) - (xi >> 1),
                                 jnp.float32)
    for _ in range(2):
        y = y * (1.5 - 0.5 * x * y * y)
    return y


def _ln_16rows(in_ref, out_ref, gamma_ref, beta_ref, idx_ref, ibase, gbase):
    gs = [gamma_ref[pl.ds(16 * k, 16)] for k in range(4)]
    bs = [beta_ref[pl.ds(16 * k, 16)] for k in range(4)]
    pvv = (idx_ref[pl.ds(ibase + gbase, 16)] & 1) << 6
    NR = 8
    for k4 in range(16 // NR):
        rows = [gbase + NR * k4 + k for k in range(NR)]
        ofss = [pvv[NR * k4 + k] for k in range(NR)]
        vss = [[in_ref[rows[k], pl.ds(ofss[k] + 16 * q, 16)]
                for q in range(4)] for k in range(NR)]
        ss = [(vs[0] + vs[1]) + (vs[2] + vs[3]) for vs in vss]
        sqs = [(vs[0] * vs[0] + vs[1] * vs[1])
               + (vs[2] * vs[2] + vs[3] * vs[3]) for vs in vss]
        tots = [plsc.cumsum(s)[15] for s in ss]
        tot2s = [plsc.cumsum(sq)[15] for sq in sqs]
        c1s, c2s = [], []
        for k in range(NR):
            mean = tots[k] * (1.0 / EMB_DIM)
            var = tot2s[k] * (1.0 / EMB_DIM) - mean * mean
            rstd = _rsqrt(var + EPS)
            c1s.append(rstd)
            c2s.append(-mean * rstd)
        for k in range(NR):
            for q in range(4):
                out_ref[rows[k], pl.ds(16 * q, 16)] = (
                    (vss[k][q] * c1s[k] + c2s[k]) * gs[q] + bs[q])


def _make_kernel():
    mesh = plsc.VectorSubcoreMesh(core_axis_name="c", subcore_axis_name="s")

    @functools.partial(
        pl.kernel,
        mesh=mesh,
        out_type=jax.ShapeDtypeStruct((B * L, EMB_DIM), jnp.float32),
        compiler_params=pltpu.CompilerParams(
            use_tc_tiling_on_sc=False,
            needs_layout_passes=False,
        ),
        scratch_types=[
            pltpu.VMEM((NB * L,), jnp.int32),
            pltpu.VMEM((L, 128), jnp.float32),
            pltpu.VMEM((L, 128), jnp.float32),
            pltpu.VMEM((L, EMB_DIM), jnp.float32),
            pltpu.VMEM((L, EMB_DIM), jnp.float32),
            pltpu.VMEM((EMB_DIM,), jnp.float32),
            pltpu.VMEM((EMB_DIM,), jnp.float32),
            pltpu.SemaphoreType.DMA,
            pltpu.SemaphoreType.DMA,
            pltpu.SemaphoreType.DMA,
            pltpu.SemaphoreType.DMA,
        ],
    )
    def kern(ids_hbm, table_hbm, gamma_hbm, beta_hbm, out_hbm,
             idx_v, in0, in1, out0, out1, gamma_v, beta_v,
             gsem0, gsem1, osem0, osem1):
        wid = lax.axis_index("s") * 2 + lax.axis_index("c")
        wrow = wid * NB * L

        pltpu.sync_copy(gamma_hbm, gamma_v)
        pltpu.sync_copy(beta_hbm, beta_v)
        pltpu.sync_copy(ids_hbm.at[wid], idx_v)

        ins = (in0, in1)
        outs = (out0, out1)
        gsems = (gsem0, gsem1)
        osems = (osem0, osem1)

        def gather_start(c, b):
            for g in GBASES:
                gidx = idx_v[pl.ds(c * L + g, 16)] >> 1
                pltpu.async_copy(table_hbm.at[gidx],
                                 ins[b].at[pl.ds(g, 16)], gsems[b])

        def gather_wait(c, b):
            for g in GBASES:
                gidx = idx_v[pl.ds(c * L + g, 16)] >> 1
                pltpu.make_async_copy(table_hbm.at[gidx],
                                      ins[b].at[pl.ds(g, 16)],
                                      gsems[b]).wait()

        def out_start(c, b):
            pltpu.async_copy(outs[b],
                             out_hbm.at[pl.ds(wrow + c * L, L)], osems[b])

        def out_wait(c, b):
            pltpu.make_async_copy(outs[b],
                                  out_hbm.at[pl.ds(wrow + c * L, L)],
                                  osems[b]).wait()

        gather_start(0, 0)
        gather_start(1, 1)

        def body(i, carry):
            for b in range(2):
                c = 2 * i + b
                gather_wait(c, b)

                @pl.when(c >= 2)
                def _():
                    out_wait(c - 2, b)

                def grp(gi, carry2):
                    _ln_16rows(ins[b], outs[b], gamma_v, beta_v,
                               idx_v, c * L, gi * 16)
                    return carry2

                lax.fori_loop(0, 12, grp, 0)
                _ln_16rows(ins[b], outs[b], gamma_v, beta_v,
                           idx_v, c * L, 184)

                out_start(c, b)

                @pl.when(c + 2 < NB)
                def _():
                    gather_start(c + 2, b)
            return carry

        lax.fori_loop(0, NB // 2, body, 0)

        out_wait(NB - 2, 0)
        out_wait(NB - 1, 1)

    return kern


_KERNEL = _make_kernel()


@jax.jit
def kernel(input_ids, table, ln_gamma, ln_beta):
    ids = input_ids.reshape(NW, NB * L)
    tv = table.reshape(TROWS, 128)
    out = _KERNEL(ids, tv, ln_gamma, ln_beta)
    return out.reshape(B, L, EMB_DIM)
